# Initial kernel scaffold; baseline (speedup 1.0000x reference)
import functools

import jax
import jax.numpy as jnp
from jax import lax
from jax.experimental import pallas as pl
from jax.experimental.pallas import tpu as pltpu
from jax.experimental.pallas import tpu_sc as plsc

B, C, H, W = 2, 128, 192, 256
WIN = 5
STRIDE = 4
GH, GW = 48, 64
P = WIN * WIN
M = 5000
HP, WP = H + 2, W + 2
TROWS = B * HP * WP

NCORES, NSUB = 2, 16
NW = NCORES * NSUB
CHUNK = 128
NCHUNK = 31
ROWS_PER_TILE = CHUNK * NCHUNK
R = NW * ROWS_PER_TILE
ROWS = M * P

_mesh = plsc.VectorSubcoreMesh(core_axis_name="c", subcore_axis_name="s")


@functools.partial(
    pl.kernel,
    out_type=(
        jax.ShapeDtypeStruct((R, C), jnp.float32),
        jax.ShapeDtypeStruct((R, C), jnp.float32),
    ),
    mesh=_mesh,
    scratch_types=[
        pltpu.VMEM((M,), jnp.int32),
        pltpu.VMEM((M,), jnp.int32),
        pltpu.VMEM((M,), jnp.int32),
        pltpu.VMEM((NCHUNK, CHUNK), jnp.int32),
        pltpu.VMEM((CHUNK, C), jnp.float32),
        pltpu.SemaphoreType.DMA,
    ],
)
def _sc_gather(t0_hbm, t1_hbm, b_hbm, i_hbm, j_hbm, out0, out1,
               b_v, i_v, j_v, idx_v, buf, sem):
    wid = lax.axis_index("s") * NCORES + lax.axis_index("c")
    base = wid * ROWS_PER_TILE

    pltpu.sync_copy(b_hbm, b_v)
    pltpu.sync_copy(i_hbm, i_v)
    pltpu.sync_copy(j_hbm, j_v)

    lanes = lax.iota(jnp.int32, 16)

    def run_feat(ids_v, table, out):
        @pl.loop(0, NCHUNK)
        def _(ci_chunk):
            for g in range(CHUNK // 16):
                r = base + ci_chunk * CHUNK + g * 16 + lanes
                k = r // P
                p = r - k * P
                valid = k < M
                ks = jnp.where(valid, k, 0)
                bb = plsc.load_gather(b_v, [ks])
                ii = plsc.load_gather(ids_v, [ks])
                gi = ii // GW
                gj = ii - gi * GW
                di = p // WIN
                dj = p - di * WIN
                row = (bb * HP + gi * STRIDE + di) * WP + gj * STRIDE + dj
                idx_v[ci_chunk, pl.ds(g * 16, 16)] = jnp.where(valid, row, 0)

        @pl.loop(0, NCHUNK)
        def _(ci_chunk):
            pltpu.async_copy(table.at[idx_v.at[ci_chunk]], buf, sem).wait()
            pltpu.sync_copy(buf, out.at[pl.ds(base + ci_chunk * CHUNK, CHUNK)])

    run_feat(i_v, t0_hbm, out0)
    run_feat(j_v, t1_hbm, out1)


def _prep(feat):
    t = jnp.transpose(feat, (0, 2, 3, 1))
    t = jnp.pad(t, ((0, 0), (2, 0), (2, 0), (0, 0)))
    return t.reshape(TROWS, C)


def kernel(feat_f0, feat_f1, hw0_f, hw0_c, b_ids, i_ids, j_ids):
    t0 = _prep(feat_f0)
    t1 = _prep(feat_f1)
    b = b_ids.astype(jnp.int32)
    i = i_ids.astype(jnp.int32)
    j = j_ids.astype(jnp.int32)
    g0, g1 = _sc_gather(t0, t1, b, i, j)

    def finish(g):
        nat = g[:ROWS].reshape(M, P, C)
        return nat.transpose(0, 2, 1).reshape(M, P, C)

    return finish(g0), finish(g1)


# speedup vs baseline: 10.7286x; 10.7286x over previous
import functools

import jax
import jax.numpy as jnp
from jax import lax
from jax.experimental import pallas as pl
from jax.experimental.pallas import tpu as pltpu
from jax.experimental.pallas import tpu_sc as plsc

B, C, H, W = 2, 128, 192, 256
WIN = 5
STRIDE = 4
GW_SHIFT, GW_MASK = 6, 63
P = WIN * WIN
M = 5000
HP, WP = H + 2, W + 2
TROWS = B * HP * WP

NCORES, NSUB = 2, 16
NW = NCORES * NSUB
MPT = 160
TILE_ROWS = MPT * P
CH = 80
NCH = TILE_ROWS // CH
R = NW * TILE_ROWS
ROWS = M * P


@functools.cache
def _build_sc_gather():
    mesh = plsc.VectorSubcoreMesh(core_axis_name="c", subcore_axis_name="s")
    return functools.partial(
        pl.kernel,
        out_type=(
            jax.ShapeDtypeStruct((R, C), jnp.float32),
            jax.ShapeDtypeStruct((R, C), jnp.float32),
        ),
        mesh=mesh,
        compiler_params=pltpu.CompilerParams(needs_layout_passes=False),
        scratch_types=[
            pltpu.VMEM((M,), jnp.int32),
            pltpu.VMEM((M,), jnp.int32),
            pltpu.VMEM((M,), jnp.int32),
            pltpu.VMEM((TILE_ROWS,), jnp.int32),
            pltpu.VMEM((CH, C), jnp.float32),
            pltpu.SemaphoreType.DMA,
        ],
    )(_sc_gather_body)


def _sc_gather_body(t0_hbm, t1_hbm, b_hbm, i_hbm, j_hbm, out0, out1,
                    b_v, i_v, j_v, idx_v, buf, sem):
    wid = lax.axis_index("s") * NCORES + lax.axis_index("c")
    base = wid * TILE_ROWS
    m0 = wid * MPT

    pltpu.sync_copy(b_hbm, b_v)
    pltpu.sync_copy(i_hbm, i_v)
    pltpu.sync_copy(j_hbm, j_v)

    def run_feat(ids_v, table, out):
        @pl.loop(0, MPT // 16)
        def _(g):
            krel = g * 16 + lax.iota(jnp.int32, 16)
            mk = jnp.minimum(m0 + krel, M - 1)
            bb = plsc.load_gather(b_v, [mk])
            ii = plsc.load_gather(ids_v, [mk])
            gi = lax.shift_right_logical(ii, GW_SHIFT)
            gj = ii & GW_MASK
            brow = (bb * HP + gi * STRIDE) * WP + gj * STRIDE
            q0 = krel * P
            for p in range(P):
                row = brow + (p // WIN) * WP + (p % WIN)
                plsc.store_scatter(idx_v, [q0 + p], row)

        @pl.loop(0, NCH)
        def _(ci):
            pltpu.async_copy(table.at[idx_v.at[pl.ds(ci * CH, CH)]],
                             buf, sem).wait()
            pltpu.sync_copy(buf, out.at[pl.ds(base + ci * CH, CH)])

    run_feat(i_v, t0_hbm, out0)
    run_feat(j_v, t1_hbm, out1)


def _prep(feat):
    t = jnp.transpose(feat, (0, 2, 3, 1))
    t = jnp.pad(t, ((0, 0), (2, 0), (2, 0), (0, 0)))
    return t.reshape(TROWS, C)


def kernel(feat_f0, feat_f1, hw0_f, hw0_c, b_ids, i_ids, j_ids):
    t0 = _prep(feat_f0)
    t1 = _prep(feat_f1)
    b = b_ids.astype(jnp.int32)
    i = i_ids.astype(jnp.int32)
    j = j_ids.astype(jnp.int32)
    g0, g1 = _build_sc_gather()(t0, t1, b, i, j)

    def finish(g):
        nat = g[:ROWS].reshape(M, P, C)
        return nat.transpose(0, 2, 1).reshape(M, P, C)

    return finish(g0), finish(g1)
